# Initial kernel scaffold; baseline (speedup 1.0000x reference)
import jax
import jax.numpy as jnp
from jax.experimental import pallas as pl


def kernel(features, p1_Wl, p1_Wr, p1_a, p1_b, p2_Wl, p2_Wr, p2_a, p2_b, v1_Wl, v1_Wr, v1_a, v1_b, v2_Wl, v2_Wr, v2_a, v2_b):
    raise NotImplementedError("write your pallas kernel here")


# speedup vs baseline: 1.3729x; 1.3729x over previous
import functools

import jax
import jax.numpy as jnp
from jax.experimental import pallas as pl
from jax.experimental.pallas import tpu as pltpu


def _gatv2_layer(x, Wl, Wr, aRow, bRow):
    N = x.shape[0]
    D = Wl.shape[1]
    hl = jnp.dot(x, Wl, preferred_element_type=jnp.float32)
    hr = jnp.dot(x, Wr, preferred_element_type=jnp.float32)
    hrT = hr.T
    aCol = aRow.reshape(D, 1)
    sl = jnp.dot(hl, aCol, preferred_element_type=jnp.float32)
    sr = jnp.dot(aRow, hrT, preferred_element_type=jnp.float32)
    z = hl[:, :, None] + hrT[None, :, :]
    P = jnp.sum(jnp.maximum(z, 0.0) * aRow.reshape(1, D, 1), axis=1)
    scores = 0.2 * (sl + sr) + 0.8 * P
    m = jnp.max(scores, axis=1, keepdims=True)
    p = jnp.exp(scores - m)
    alpha = p / jnp.sum(p, axis=1, keepdims=True)
    return jnp.dot(alpha, hr, preferred_element_type=jnp.float32) + bRow


def _net_kernel(x_ref, W1l_ref, W1r_ref, a1_ref, b1_ref,
                W2l_ref, W2r_ref, a2_ref, b2_ref, out_ref):
    x = x_ref[0]
    h = jnp.tanh(_gatv2_layer(x, W1l_ref[0], W1r_ref[0], a1_ref[0], b1_ref[0]))
    g = jnp.tanh(_gatv2_layer(h, W2l_ref[0], W2r_ref[0], a2_ref[0], b2_ref[0]))
    out_ref[0, 0, 0, :] = jnp.mean(g, axis=0)


@jax.jit
def kernel(features, p1_Wl, p1_Wr, p1_a, p1_b, p2_Wl, p2_Wr, p2_a, p2_b,
           v1_Wl, v1_Wr, v1_a, v1_b, v2_Wl, v2_Wr, v2_a, v2_b):
    B, N, F = features.shape
    D = p1_Wl.shape[1]
    W1l = jnp.stack([p1_Wl, v1_Wl])
    W1r = jnp.stack([p1_Wr, v1_Wr])
    a1 = jnp.stack([p1_a, v1_a])[:, None, :]
    b1 = jnp.stack([p1_b, v1_b])[:, None, :]
    W2l = jnp.stack([p2_Wl, v2_Wl])
    W2r = jnp.stack([p2_Wr, v2_Wr])
    a2 = jnp.stack([p2_a, v2_a])[:, None, :]
    b2 = jnp.stack([p2_b, v2_b])[:, None, :]

    wspec1 = pl.BlockSpec((1, F, D), lambda n, b: (n, 0, 0))
    wspec2 = pl.BlockSpec((1, D, D), lambda n, b: (n, 0, 0))
    vspec = pl.BlockSpec((1, 1, D), lambda n, b: (n, 0, 0))
    out = pl.pallas_call(
        _net_kernel,
        grid=(2, B),
        in_specs=[
            pl.BlockSpec((1, N, F), lambda n, b: (b, 0, 0)),
            wspec1, wspec1, vspec, vspec,
            wspec2, wspec2, vspec, vspec,
        ],
        out_specs=pl.BlockSpec((1, 1, 1, D), lambda n, b: (n, b, 0, 0)),
        out_shape=jax.ShapeDtypeStruct((2, B, 1, D), jnp.float32),
        compiler_params=pltpu.CompilerParams(
            dimension_semantics=("parallel", "parallel")),
    )(features, W1l, W1r, a1, b1, W2l, W2r, a2, b2)
    return (out[0, :, 0, :], out[1, :, 0, :])


# speedup vs baseline: 2.1092x; 1.5363x over previous
import jax
import jax.numpy as jnp
from jax.experimental import pallas as pl
from jax.experimental.pallas import tpu as pltpu

_BG = 4


def _gatv2_layer(x, Wl, Wr, aRow, bRow):
    N = x.shape[0]
    D = Wl.shape[1]
    hl = jnp.dot(x, Wl, preferred_element_type=jnp.float32)
    hr = jnp.dot(x, Wr, preferred_element_type=jnp.float32)
    aCol = aRow.reshape(D, 1)
    sl = jnp.dot(hl, aCol, preferred_element_type=jnp.float32)
    sr = jnp.dot(aRow, hr.T, preferred_element_type=jnp.float32)
    hlb = hl.astype(jnp.bfloat16)
    hrTb = hr.T.astype(jnp.bfloat16)
    ab = aRow.astype(jnp.bfloat16)
    parts = []
    for d0 in range(0, D, D // 2):
        acc = jnp.zeros((N, N), jnp.bfloat16)
        for d in range(d0, d0 + D // 2):
            z = hlb[:, d:d + 1] + hrTb[d:d + 1, :]
            acc = acc + jnp.maximum(z, 0) * ab[:, d:d + 1]
        parts.append(acc)
    P = (parts[0] + parts[1]).astype(jnp.float32)
    scores = 0.2 * (sl + sr) + 0.8 * P
    m = jnp.max(scores, axis=1, keepdims=True)
    p = jnp.exp(scores - m)
    alpha = p / jnp.sum(p, axis=1, keepdims=True)
    return jnp.dot(alpha, hr, preferred_element_type=jnp.float32) + bRow


def _net_kernel(x_ref, W1l_ref, W1r_ref, a1_ref, b1_ref,
                W2l_ref, W2r_ref, a2_ref, b2_ref, out_ref):
    for g in range(_BG):
        x = x_ref[g]
        h = jnp.tanh(_gatv2_layer(x, W1l_ref[0], W1r_ref[0],
                                  a1_ref[0], b1_ref[0]))
        g2 = jnp.tanh(_gatv2_layer(h, W2l_ref[0], W2r_ref[0],
                                   a2_ref[0], b2_ref[0]))
        out_ref[0, g, 0, :] = jnp.mean(g2, axis=0)


@jax.jit
def kernel(features, p1_Wl, p1_Wr, p1_a, p1_b, p2_Wl, p2_Wr, p2_a, p2_b,
           v1_Wl, v1_Wr, v1_a, v1_b, v2_Wl, v2_Wr, v2_a, v2_b):
    B, N, F = features.shape
    D = p1_Wl.shape[1]
    W1l = jnp.stack([p1_Wl, v1_Wl])
    W1r = jnp.stack([p1_Wr, v1_Wr])
    a1 = jnp.stack([p1_a, v1_a])[:, None, :]
    b1 = jnp.stack([p1_b, v1_b])[:, None, :]
    W2l = jnp.stack([p2_Wl, v2_Wl])
    W2r = jnp.stack([p2_Wr, v2_Wr])
    a2 = jnp.stack([p2_a, v2_a])[:, None, :]
    b2 = jnp.stack([p2_b, v2_b])[:, None, :]

    wspec1 = pl.BlockSpec((1, F, D), lambda n, c: (n, 0, 0))
    wspec2 = pl.BlockSpec((1, D, D), lambda n, c: (n, 0, 0))
    vspec = pl.BlockSpec((1, 1, D), lambda n, c: (n, 0, 0))
    out = pl.pallas_call(
        _net_kernel,
        grid=(2, B // _BG),
        in_specs=[
            pl.BlockSpec((_BG, N, F), lambda n, c: (c, 0, 0)),
            wspec1, wspec1, vspec, vspec,
            wspec2, wspec2, vspec, vspec,
        ],
        out_specs=pl.BlockSpec((1, _BG, 1, D), lambda n, c: (n, c, 0, 0)),
        out_shape=jax.ShapeDtypeStruct((2, B, 1, D), jnp.float32),
        compiler_params=pltpu.CompilerParams(
            dimension_semantics=("parallel", "parallel")),
    )(features, W1l, W1r, a1, b1, W2l, W2r, a2, b2)
    return (out[0, :, 0, :], out[1, :, 0, :])


# speedup vs baseline: 2.2354x; 1.0598x over previous
import jax
import jax.numpy as jnp
from jax.experimental import pallas as pl
from jax.experimental.pallas import tpu as pltpu

_BG = 4


def _gatv2_layer(x, Wl, Wr, aRow, bRow):
    N = x.shape[0]
    D = Wl.shape[1]
    hl = jnp.dot(x, Wl, preferred_element_type=jnp.float32)
    hr = jnp.dot(x, Wr, preferred_element_type=jnp.float32)
    sr = 0.2 * jnp.dot(aRow, hr.T, preferred_element_type=jnp.float32)
    hlb = hl.astype(jnp.bfloat16)
    hrTb = hr.T.astype(jnp.bfloat16)
    ab = (0.8 * aRow).astype(jnp.bfloat16)
    parts = []
    for d0 in range(0, D, D // 2):
        acc = jnp.zeros((N, N), jnp.bfloat16)
        for d in range(d0, d0 + D // 2):
            z = hlb[:, d:d + 1] + hrTb[d:d + 1, :]
            acc = acc + jnp.maximum(z, 0) * ab[:, d:d + 1]
        parts.append(acc)
    scores = (parts[0] + parts[1]).astype(jnp.float32) + sr
    m = jnp.max(scores, axis=1, keepdims=True)
    p = jnp.exp(scores - m)
    s = jnp.sum(p, axis=1, keepdims=True)
    return jnp.dot(p, hr, preferred_element_type=jnp.float32) / s + bRow


def _net_kernel(x_ref, W1l_ref, W1r_ref, a1_ref, b1_ref,
                W2l_ref, W2r_ref, a2_ref, b2_ref, out_ref):
    for g in range(_BG):
        x = x_ref[g]
        h = jnp.tanh(_gatv2_layer(x, W1l_ref[0], W1r_ref[0],
                                  a1_ref[0], b1_ref[0]))
        g2 = jnp.tanh(_gatv2_layer(h, W2l_ref[0], W2r_ref[0],
                                   a2_ref[0], b2_ref[0]))
        out_ref[0, g, 0, :] = jnp.mean(g2, axis=0)


@jax.jit
def kernel(features, p1_Wl, p1_Wr, p1_a, p1_b, p2_Wl, p2_Wr, p2_a, p2_b,
           v1_Wl, v1_Wr, v1_a, v1_b, v2_Wl, v2_Wr, v2_a, v2_b):
    B, N, F = features.shape
    D = p1_Wl.shape[1]
    W1l = jnp.stack([p1_Wl, v1_Wl])
    W1r = jnp.stack([p1_Wr, v1_Wr])
    a1 = jnp.stack([p1_a, v1_a])[:, None, :]
    b1 = jnp.stack([p1_b, v1_b])[:, None, :]
    W2l = jnp.stack([p2_Wl, v2_Wl])
    W2r = jnp.stack([p2_Wr, v2_Wr])
    a2 = jnp.stack([p2_a, v2_a])[:, None, :]
    b2 = jnp.stack([p2_b, v2_b])[:, None, :]

    wspec1 = pl.BlockSpec((1, F, D), lambda n, c: (n, 0, 0))
    wspec2 = pl.BlockSpec((1, D, D), lambda n, c: (n, 0, 0))
    vspec = pl.BlockSpec((1, 1, D), lambda n, c: (n, 0, 0))
    out = pl.pallas_call(
        _net_kernel,
        grid=(2, B // _BG),
        in_specs=[
            pl.BlockSpec((_BG, N, F), lambda n, c: (c, 0, 0)),
            wspec1, wspec1, vspec, vspec,
            wspec2, wspec2, vspec, vspec,
        ],
        out_specs=pl.BlockSpec((1, _BG, 1, D), lambda n, c: (n, c, 0, 0)),
        out_shape=jax.ShapeDtypeStruct((2, B, 1, D), jnp.float32),
        compiler_params=pltpu.CompilerParams(
            dimension_semantics=("parallel", "parallel")),
    )(features, W1l, W1r, a1, b1, W2l, W2r, a2, b2)
    return (out[0, :, 0, :], out[1, :, 0, :])


# speedup vs baseline: 2.2565x; 1.0094x over previous
import jax
import jax.numpy as jnp
from jax.experimental import pallas as pl
from jax.experimental.pallas import tpu as pltpu

_BG = 4


def _gatv2_layer(x, Wl, Wr, aRow, bRow):
    N = x.shape[0]
    D = Wl.shape[1]
    hl = jnp.dot(x, Wl, preferred_element_type=jnp.float32)
    hr = jnp.dot(x, Wr, preferred_element_type=jnp.float32)
    sr = 0.2 * jnp.dot(aRow, hr.T, preferred_element_type=jnp.float32)
    hlb = hl.astype(jnp.bfloat16)
    hrTb = hr.T.astype(jnp.bfloat16)
    ab = (0.8 * aRow).astype(jnp.bfloat16)
    parts = []
    for d0 in range(0, D, D // 2):
        acc = jnp.zeros((N, N), jnp.bfloat16)
        for d in range(d0, d0 + D // 2):
            z = hlb[:, d:d + 1] + hrTb[d:d + 1, :]
            acc = acc + jnp.maximum(z, 0) * ab[:, d:d + 1]
        parts.append(acc)
    scores = (parts[0] + parts[1]).astype(jnp.float32) + sr
    m = jnp.max(scores, axis=1, keepdims=True)
    p = jnp.exp(scores - m)
    s = jnp.sum(p, axis=1, keepdims=True)
    return jnp.dot(p, hr, preferred_element_type=jnp.float32) / s + bRow


def _net_kernel(x_ref, W1l_ref, W1r_ref, a1_ref, b1_ref,
                W2l_ref, W2r_ref, a2_ref, b2_ref, out_ref):
    for g in range(_BG):
        x = x_ref[g]
        for n in range(2):
            h = jnp.tanh(_gatv2_layer(x, W1l_ref[n], W1r_ref[n],
                                      a1_ref[n], b1_ref[n]))
            g2 = jnp.tanh(_gatv2_layer(h, W2l_ref[n], W2r_ref[n],
                                       a2_ref[n], b2_ref[n]))
            out_ref[n, g, 0, :] = jnp.mean(g2, axis=0)


@jax.jit
def kernel(features, p1_Wl, p1_Wr, p1_a, p1_b, p2_Wl, p2_Wr, p2_a, p2_b,
           v1_Wl, v1_Wr, v1_a, v1_b, v2_Wl, v2_Wr, v2_a, v2_b):
    B, N, F = features.shape
    D = p1_Wl.shape[1]
    W1l = jnp.stack([p1_Wl, v1_Wl])
    W1r = jnp.stack([p1_Wr, v1_Wr])
    a1 = jnp.stack([p1_a, v1_a])[:, None, :]
    b1 = jnp.stack([p1_b, v1_b])[:, None, :]
    W2l = jnp.stack([p2_Wl, v2_Wl])
    W2r = jnp.stack([p2_Wr, v2_Wr])
    a2 = jnp.stack([p2_a, v2_a])[:, None, :]
    b2 = jnp.stack([p2_b, v2_b])[:, None, :]

    wspec1 = pl.BlockSpec((2, F, D), lambda c: (0, 0, 0))
    wspec2 = pl.BlockSpec((2, D, D), lambda c: (0, 0, 0))
    vspec = pl.BlockSpec((2, 1, D), lambda c: (0, 0, 0))
    out = pl.pallas_call(
        _net_kernel,
        grid=(B // _BG,),
        in_specs=[
            pl.BlockSpec((_BG, N, F), lambda c: (c, 0, 0)),
            wspec1, wspec1, vspec, vspec,
            wspec2, wspec2, vspec, vspec,
        ],
        out_specs=pl.BlockSpec((2, _BG, 1, D), lambda c: (0, c, 0, 0)),
        out_shape=jax.ShapeDtypeStruct((2, B, 1, D), jnp.float32),
        compiler_params=pltpu.CompilerParams(
            dimension_semantics=("parallel",)),
    )(features, W1l, W1r, a1, b1, W2l, W2r, a2, b2)
    return (out[0, :, 0, :], out[1, :, 0, :])
